# baseline (speedup 1.0000x reference)
import functools

import jax
import jax.numpy as jnp
from jax import lax
from jax.experimental import pallas as pl
from jax.experimental.pallas import tpu as pltpu
from jax.experimental.pallas import tpu_sc as plsc

N_NODES = 10000
N_EDGES = 160000
D_IN = 256
H = 512
C_OUT = 18
N_LAYERS = 9
EPS = 1e-5

NP = 10240
NT = NP // 512
NC = 2
NS = 16
EB = 128
NBLK = 80
E_PAD = NS * NBLK * EB
ROWS_PER_SUB = NP // NS
NCH = 4

_f32 = jnp.float32
_i32 = jnp.int32


def _sc_mesh():
    return plsc.VectorSubcoreMesh(core_axis_name="c", subcore_axis_name="s")


def _deg_body(colidx_hbm, ones_hbm, zeros_hbm, deg_hbm, col_v, buf_v, acc_sh):
    core = lax.axis_index("c")
    sid = lax.axis_index("s")
    pltpu.sync_copy(zeros_hbm, acc_sh.at[pl.ds(sid * ROWS_PER_SUB, ROWS_PER_SUB)])
    pltpu.sync_copy(colidx_hbm.at[sid], col_v)
    pltpu.sync_copy(ones_hbm, buf_v)
    plsc.subcore_barrier()

    half = NBLK // NC

    def blk(j, _):
        pltpu.sync_copy(buf_v, acc_sh.at[col_v.at[core * half + j]], add=True)
        return _

    lax.fori_loop(0, half, blk, None)
    plsc.subcore_barrier()
    pltpu.sync_copy(acc_sh.at[pl.ds(sid * ROWS_PER_SUB, ROWS_PER_SUB)],
                    deg_hbm.at[core, pl.ds(sid * ROWS_PER_SUB, ROWS_PER_SUB)])


def _sc_degree(colidx, ones_b, zeros_b):
    return pl.kernel(
        _deg_body,
        out_type=jax.ShapeDtypeStruct((NC, NP, 128), _f32),
        mesh=_sc_mesh(),
        scratch_types=[
            pltpu.VMEM((NBLK, EB), _i32),
            pltpu.VMEM((EB, 128), _f32),
            pltpu.VMEM_SHARED((NP, 128), _f32),
        ],
    )(colidx, ones_b, zeros_b)


HBLK = NBLK // 2


def _agg_body(g2_hbm, rowidx_hbm, colidx_hbm, zeros_hbm, s_hbm,
              row_v, col_v, buf0_v, buf1_v, gs0, gs1, acc_sh):
    core = lax.axis_index("c")
    sid = lax.axis_index("s")
    for p in range(2):
        chunk = core * 2 + p
        pltpu.sync_copy(zeros_hbm, acc_sh.at[pl.ds(sid * ROWS_PER_SUB, ROWS_PER_SUB)])
        plsc.subcore_barrier()

        for q in range(NBLK // HBLK):
            pltpu.sync_copy(rowidx_hbm.at[chunk, sid, pl.ds(q * HBLK, HBLK)],
                            row_v)
            pltpu.sync_copy(colidx_hbm.at[sid, pl.ds(q * HBLK, HBLK)], col_v)
            pltpu.async_copy(g2_hbm.at[row_v.at[0]], buf0_v, gs0)

            def blk(jj, _):
                j = 2 * jj
                pltpu.async_copy(g2_hbm.at[row_v.at[j + 1]], buf1_v, gs1)
                pltpu.make_async_copy(g2_hbm.at[row_v.at[j]], buf0_v,
                                      gs0).wait()
                pltpu.sync_copy(buf0_v, acc_sh.at[col_v.at[j]], add=True)

                @pl.when(jj + 1 < HBLK // 2)
                def _():
                    pltpu.async_copy(g2_hbm.at[row_v.at[j + 2]], buf0_v, gs0)

                pltpu.make_async_copy(g2_hbm.at[row_v.at[j + 1]], buf1_v,
                                      gs1).wait()
                pltpu.sync_copy(buf1_v, acc_sh.at[col_v.at[j + 1]], add=True)
                return _

            lax.fori_loop(0, HBLK // 2, blk, None)
        plsc.subcore_barrier()
        pltpu.sync_copy(acc_sh.at[pl.ds(sid * ROWS_PER_SUB, ROWS_PER_SUB)],
                        s_hbm.at[chunk, pl.ds(sid * ROWS_PER_SUB, ROWS_PER_SUB)])
        plsc.subcore_barrier()


def _sc_aggregate(g_chunks, rowidx4, colidx, zeros_b):
    g2 = g_chunks.reshape(NCH * NP, 128)
    return pl.kernel(
        _agg_body,
        out_type=jax.ShapeDtypeStruct((NCH, NP, 128), _f32),
        mesh=_sc_mesh(),
        scratch_types=[
            pltpu.VMEM((HBLK, EB), _i32),
            pltpu.VMEM((HBLK, EB), _i32),
            pltpu.VMEM((EB, 128), _f32),
            pltpu.VMEM((EB, 128), _f32),
            pltpu.SemaphoreType.DMA,
            pltpu.SemaphoreType.DMA,
            pltpu.VMEM_SHARED((NP, 128), _f32),
        ],
    )(g2, rowidx4, colidx, zeros_b)


def _dis_kernel(deg_ref, dis_ref):
    dis_ref[...] = lax.rsqrt(deg_ref[0] + deg_ref[1] + 1.0)


def _tc_dis(deg):
    return pl.pallas_call(
        _dis_kernel,
        grid=(NT,),
        in_specs=[pl.BlockSpec((NC, 512, 128), lambda n: (0, n, 0))],
        out_specs=pl.BlockSpec((512, 128), lambda n: (n, 0)),
        out_shape=jax.ShapeDtypeStruct((NP, 128), _f32),
    )(deg)


def _in_mm_kernel(x_ref, w_ref, b_ref, h_ref, acc_ref):
    kc = pl.program_id(1)

    @pl.when(kc == 0)
    def _():
        acc_ref[...] = jnp.zeros_like(acc_ref)

    acc_ref[...] += jnp.dot(x_ref[...], w_ref[...],
                            preferred_element_type=_f32)

    @pl.when(kc == (D_IN // 128) - 1)
    def _():
        h_ref[...] = acc_ref[...] + b_ref[0:1, :]


def _tc_in_matmul(xp, W_in, b_in2):
    return pl.pallas_call(
        _in_mm_kernel,
        grid=(NT, D_IN // 128),
        in_specs=[
            pl.BlockSpec((512, 128), lambda n, k: (n, k)),
            pl.BlockSpec((128, H), lambda n, k: (k, 0)),
            pl.BlockSpec((8, H), lambda n, k: (0, 0)),
        ],
        out_specs=pl.BlockSpec((512, H), lambda n, k: (n, 0)),
        out_shape=jax.ShapeDtypeStruct((NP, H), _f32),
        scratch_shapes=[pltpu.VMEM((512, H), _f32)],
        compiler_params=pltpu.CompilerParams(
            dimension_semantics=("parallel", "arbitrary")),
    )(xp, W_in, b_in2)


def _g_mm_kernel(h_ref, wc_ref, dis_ref, g_ref, acc_ref):
    kc = pl.program_id(1)

    @pl.when(kc == 0)
    def _():
        acc_ref[...] = jnp.zeros_like(acc_ref)

    acc_ref[...] += jnp.dot(h_ref[...], wc_ref[...],
                            preferred_element_type=_f32)

    @pl.when(kc == NCH - 1)
    def _():
        dis_col = dis_ref[:, 0:1]
        for c in range(NCH):
            g_ref[c] = dis_col * acc_ref[:, c * 128:(c + 1) * 128]


def _r_mm_kernel(h_ref, wr_ref, r_ref, acc_ref):
    kc = pl.program_id(1)

    @pl.when(kc == 0)
    def _():
        acc_ref[...] = jnp.zeros_like(acc_ref)

    acc_ref[...] += jnp.dot(h_ref[...], wr_ref[...],
                            preferred_element_type=_f32)

    @pl.when(kc == NCH - 1)
    def _():
        r_ref[...] = acc_ref[...]


def _bn_relu_block(y_blk, stats_ref, bng_ref, bnb_ref):
    mean = stats_ref[0, 0:1, :] * (1.0 / N_NODES)
    var = stats_ref[0, 1:2, :] * (1.0 / N_NODES) - mean * mean
    inv = lax.rsqrt(var + EPS)
    return jnp.maximum((y_blk - mean) * (inv * bng_ref[0, 0:1, :])
                       + bnb_ref[0, 0:1, :], 0.0)


def _g_mm_fused_kernel(y_ref, stats_ref, bng_ref, bnb_ref,
                       wc_ref, dis_ref, g_ref, acc_ref):
    kc = pl.program_id(1)

    @pl.when(kc == 0)
    def _():
        acc_ref[...] = jnp.zeros_like(acc_ref)

    h_blk = _bn_relu_block(y_ref[...], stats_ref, bng_ref, bnb_ref)
    acc_ref[...] += jnp.dot(h_blk, wc_ref[...], preferred_element_type=_f32)

    @pl.when(kc == NCH - 1)
    def _():
        dis_col = dis_ref[:, 0:1]
        for c in range(NCH):
            g_ref[c] = dis_col * acc_ref[:, c * 128:(c + 1) * 128]


def _r_mm_fused_kernel(y_ref, stats_ref, bng_ref, bnb_ref,
                       wr_ref, r_ref, acc_ref):
    kc = pl.program_id(1)

    @pl.when(kc == 0)
    def _():
        acc_ref[...] = jnp.zeros_like(acc_ref)

    h_blk = _bn_relu_block(y_ref[...], stats_ref, bng_ref, bnb_ref)
    acc_ref[...] += jnp.dot(h_blk, wr_ref[...], preferred_element_type=_f32)

    @pl.when(kc == NCH - 1)
    def _():
        r_ref[...] = acc_ref[...]


_G_OUT = jax.ShapeDtypeStruct((NCH, NP, 128), _f32)
_G_OUT_SPEC = pl.BlockSpec((NCH, 512, 128), lambda n, k: (0, n, 0))
_R_OUT = jax.ShapeDtypeStruct((NP, H), _f32)
_R_OUT_SPEC = pl.BlockSpec((512, H), lambda n, k: (n, 0))
_MM_SCRATCH = [pltpu.VMEM((512, H), _f32)]
_CP = pltpu.CompilerParams(dimension_semantics=("parallel", "arbitrary"))
_H_SPEC = pl.BlockSpec((512, 128), lambda n, k: (n, k))
_W_SPEC = pl.BlockSpec((128, H), lambda n, k: (k, 0))
_DIS_SPEC = pl.BlockSpec((512, 128), lambda n, k: (n, 0))
_ST_SPEC = pl.BlockSpec((1, 8, 128), lambda n, k: (k, 0, 0))


def _tc_g_matmul(h, Wc, dis):
    return pl.pallas_call(
        _g_mm_kernel, grid=(NT, NCH),
        in_specs=[_H_SPEC, _W_SPEC, _DIS_SPEC],
        out_specs=_G_OUT_SPEC, out_shape=_G_OUT,
        scratch_shapes=_MM_SCRATCH, compiler_params=_CP,
    )(h, Wc, dis)


def _tc_r_matmul(h, Wr):
    return pl.pallas_call(
        _r_mm_kernel, grid=(NT, NCH),
        in_specs=[_H_SPEC, _W_SPEC],
        out_specs=_R_OUT_SPEC, out_shape=_R_OUT,
        scratch_shapes=_MM_SCRATCH, compiler_params=_CP,
    )(h, Wr)


def _tc_g_matmul_fused(y, stats, bng, bnb, Wc, dis):
    return pl.pallas_call(
        _g_mm_fused_kernel, grid=(NT, NCH),
        in_specs=[_H_SPEC, _ST_SPEC, _ST_SPEC, _ST_SPEC, _W_SPEC, _DIS_SPEC],
        out_specs=_G_OUT_SPEC, out_shape=_G_OUT,
        scratch_shapes=_MM_SCRATCH, compiler_params=_CP,
    )(y, stats, bng, bnb, Wc, dis)


def _tc_r_matmul_fused(y, stats, bng, bnb, Wr):
    return pl.pallas_call(
        _r_mm_fused_kernel, grid=(NT, NCH),
        in_specs=[_H_SPEC, _ST_SPEC, _ST_SPEC, _ST_SPEC, _W_SPEC],
        out_specs=_R_OUT_SPEC, out_shape=_R_OUT,
        scratch_shapes=_MM_SCRATCH, compiler_params=_CP,
    )(y, stats, bng, bnb, Wr)


def _combine_kernel(s_ref, g_ref, r_ref, dis_ref, b_ref, y_ref, stats_ref):
    n = pl.program_id(0)

    @pl.when(n == 0)
    def _():
        stats_ref[...] = jnp.zeros_like(stats_ref)

    dis_col = dis_ref[:, 0:1]
    rowid = n * 512 + lax.broadcasted_iota(_i32, (512, 1), 0)
    valid = rowid < N_NODES
    for c in range(NCH):
        ycol = (s_ref[c] + g_ref[c]) * dis_col \
            + r_ref[:, c * 128:(c + 1) * 128] + b_ref[c, 0:1, :]
        ycol = jnp.where(valid, ycol, 0.0)
        y_ref[:, c * 128:(c + 1) * 128] = ycol
        stats_ref[c, 0, :] += jnp.sum(ycol, axis=0)
        stats_ref[c, 1, :] += jnp.sum(ycol * ycol, axis=0)


def _tc_combine(S, g, r, dis, conv_b3):
    return pl.pallas_call(
        _combine_kernel,
        grid=(NT,),
        in_specs=[
            pl.BlockSpec((NCH, 512, 128), lambda n: (0, n, 0)),
            pl.BlockSpec((NCH, 512, 128), lambda n: (0, n, 0)),
            pl.BlockSpec((512, H), lambda n: (n, 0)),
            pl.BlockSpec((512, 128), lambda n: (n, 0)),
            pl.BlockSpec((NCH, 8, 128), lambda n: (0, 0, 0)),
        ],
        out_specs=(pl.BlockSpec((512, H), lambda n: (n, 0)),
                   pl.BlockSpec((NCH, 8, 128), lambda n: (0, 0, 0))),
        out_shape=(jax.ShapeDtypeStruct((NP, H), _f32),
                   jax.ShapeDtypeStruct((NCH, 8, 128), _f32)),
    )(S, g, r, dis, conv_b3)


def _head_kernel(y_ref, stats_ref, bng_ref, bnb_ref, wo_ref, bo_ref,
                 out_ref, acc_ref):
    kc = pl.program_id(1)

    @pl.when(kc == 0)
    def _():
        acc_ref[...] = jnp.zeros_like(acc_ref)

    h_blk = _bn_relu_block(y_ref[...], stats_ref, bng_ref, bnb_ref)
    acc_ref[...] += jnp.dot(h_blk, wo_ref[...], preferred_element_type=_f32)

    @pl.when(kc == NCH - 1)
    def _():
        z = acc_ref[...] + bo_ref[0:1, :]
        m = jnp.max(z, axis=1, keepdims=True)
        lse = jnp.log(jnp.sum(jnp.exp(z - m), axis=1, keepdims=True))
        out_ref[...] = z - m - lse


def _tc_head(y, stats, bng, bnb, Wo, bo):
    return pl.pallas_call(
        _head_kernel,
        grid=(NT, NCH),
        in_specs=[
            pl.BlockSpec((512, 128), lambda n, k: (n, k)),
            pl.BlockSpec((1, 8, 128), lambda n, k: (k, 0, 0)),
            pl.BlockSpec((1, 8, 128), lambda n, k: (k, 0, 0)),
            pl.BlockSpec((1, 8, 128), lambda n, k: (k, 0, 0)),
            pl.BlockSpec((128, 128), lambda n, k: (k, 0)),
            pl.BlockSpec((8, 128), lambda n, k: (0, 0)),
        ],
        out_specs=pl.BlockSpec((512, 128), lambda n, k: (n, 0)),
        out_shape=jax.ShapeDtypeStruct((NP, 128), _f32),
        scratch_shapes=[pltpu.VMEM((512, 128), _f32)],
        compiler_params=pltpu.CompilerParams(
            dimension_semantics=("parallel", "arbitrary")),
    )(y, stats, bng, bnb, Wo, bo)


def kernel(x, edge_index, W_in, b_in, conv_W, conv_b, res_W, bn_g, bn_b,
           W_out, b_out):
    row = edge_index[0].astype(_i32)
    col = edge_index[1].astype(_i32)
    pad = E_PAD - N_EDGES
    rowp = jnp.concatenate([row, jnp.zeros((pad,), _i32)])
    colp = jnp.concatenate([col, jnp.full((pad,), N_NODES, _i32)])
    col_sh = colp.reshape(NS, NBLK, EB)
    col_sh2 = colp.reshape(NS, NBLK, EB)
    rowidx4 = (rowp.reshape(NS, NBLK, EB)[None]
               + (jnp.arange(NCH, dtype=_i32) * NP)[:, None, None, None])

    zeros_b = jnp.zeros((ROWS_PER_SUB, 128), _f32)
    ones_b = jnp.ones((EB, 128), _f32)

    deg = _sc_degree(col_sh, ones_b, zeros_b)
    dis = _tc_dis(deg)

    xp = jnp.pad(x, ((0, NP - N_NODES), (0, 0)))
    b_in2 = jnp.broadcast_to(b_in, (8, H))
    conv_b3 = jnp.broadcast_to(conv_b.reshape(N_LAYERS, NCH, 1, 128),
                               (N_LAYERS, NCH, 8, 128))
    bn_g3 = jnp.broadcast_to(bn_g.reshape(N_LAYERS, NCH, 1, 128),
                             (N_LAYERS, NCH, 8, 128))
    bn_b3 = jnp.broadcast_to(bn_b.reshape(N_LAYERS, NCH, 1, 128),
                             (N_LAYERS, NCH, 8, 128))
    Wo_pad = jnp.pad(W_out, ((0, 0), (0, 128 - C_OUT)))
    bo_pad = jnp.broadcast_to(
        jnp.concatenate([b_out, jnp.full((128 - C_OUT,), -1e30, _f32)]),
        (8, 128))

    h0 = _tc_in_matmul(xp, W_in, b_in2)

    y, stats = None, None
    for i in range(N_LAYERS):
        if i == 0:
            g = _tc_g_matmul(h0, conv_W[0], dis)
        else:
            g = _tc_g_matmul_fused(y, stats, bn_g3[i - 1], bn_b3[i - 1],
                                   conv_W[i], dis)
        S = _sc_aggregate(g, rowidx4, col_sh2, zeros_b)
        if i == 0:
            r = _tc_r_matmul(h0, res_W[0])
        else:
            r = _tc_r_matmul_fused(y, stats, bn_g3[i - 1], bn_b3[i - 1],
                                   res_W[i])
        y, stats = _tc_combine(S, g, r, dis, conv_b3[i])

    out = _tc_head(y, stats, bn_g3[N_LAYERS - 1], bn_b3[N_LAYERS - 1],
                   Wo_pad, bo_pad)
    return out[:N_NODES, :C_OUT]


# speedup vs baseline: 1.0747x; 1.0747x over previous
import functools

import jax
import jax.numpy as jnp
from jax import lax
from jax.experimental import pallas as pl
from jax.experimental.pallas import tpu as pltpu
from jax.experimental.pallas import tpu_sc as plsc

N_NODES = 10000
N_EDGES = 160000
D_IN = 256
H = 512
C_OUT = 18
N_LAYERS = 9
EPS = 1e-5

NP = 10240
NT = NP // 512
NC = 2
NS = 16
EB = 128
NBLK = 80
E_PAD = NS * NBLK * EB
ROWS_PER_SUB = NP // NS
NCH = 4

_f32 = jnp.float32
_i32 = jnp.int32


def _sc_mesh():
    return plsc.VectorSubcoreMesh(core_axis_name="c", subcore_axis_name="s")


def _deg_body(colidx_hbm, ones_hbm, zeros_hbm, deg_hbm, col_v, buf_v, acc_sh):
    core = lax.axis_index("c")
    sid = lax.axis_index("s")
    pltpu.sync_copy(zeros_hbm, acc_sh.at[pl.ds(sid * ROWS_PER_SUB, ROWS_PER_SUB)])
    pltpu.sync_copy(colidx_hbm.at[sid], col_v)
    pltpu.sync_copy(ones_hbm, buf_v)
    plsc.subcore_barrier()

    half = NBLK // NC

    def blk(j, _):
        pltpu.sync_copy(buf_v, acc_sh.at[col_v.at[core * half + j]], add=True)
        return _

    lax.fori_loop(0, half, blk, None)
    plsc.subcore_barrier()
    pltpu.sync_copy(acc_sh.at[pl.ds(sid * ROWS_PER_SUB, ROWS_PER_SUB)],
                    deg_hbm.at[core, pl.ds(sid * ROWS_PER_SUB, ROWS_PER_SUB)])


def _sc_degree(colidx, ones_b, zeros_b):
    return pl.kernel(
        _deg_body,
        out_type=jax.ShapeDtypeStruct((NC, NP, 128), _f32),
        mesh=_sc_mesh(),
        scratch_types=[
            pltpu.VMEM((NBLK, EB), _i32),
            pltpu.VMEM((EB, 128), _f32),
            pltpu.VMEM_SHARED((NP, 128), _f32),
        ],
    )(colidx, ones_b, zeros_b)


HBLK = NBLK // 2


def _agg_body(g2_hbm, rowidx_hbm, colidx_hbm, zeros_hbm, s_hbm,
              row_v, col_v, buf0_v, buf1_v, gs0, gs1, acc_sh):
    core = lax.axis_index("c")
    sid = lax.axis_index("s")
    for p in range(2):
        chunk = core * 2 + p
        pltpu.sync_copy(zeros_hbm, acc_sh.at[pl.ds(sid * ROWS_PER_SUB, ROWS_PER_SUB)])
        plsc.subcore_barrier()

        for q in range(NBLK // HBLK):
            pltpu.sync_copy(rowidx_hbm.at[chunk, sid, pl.ds(q * HBLK, HBLK)],
                            row_v)
            pltpu.sync_copy(colidx_hbm.at[sid, pl.ds(q * HBLK, HBLK)], col_v)
            pltpu.async_copy(g2_hbm.at[row_v.at[0]], buf0_v, gs0)

            def blk(jj, _):
                j = 2 * jj
                pltpu.async_copy(g2_hbm.at[row_v.at[j + 1]], buf1_v, gs1)
                pltpu.make_async_copy(g2_hbm.at[row_v.at[j]], buf0_v,
                                      gs0).wait()
                pltpu.sync_copy(buf0_v, acc_sh.at[col_v.at[j]], add=True)

                @pl.when(jj + 1 < HBLK // 2)
                def _():
                    pltpu.async_copy(g2_hbm.at[row_v.at[j + 2]], buf0_v, gs0)

                pltpu.make_async_copy(g2_hbm.at[row_v.at[j + 1]], buf1_v,
                                      gs1).wait()
                pltpu.sync_copy(buf1_v, acc_sh.at[col_v.at[j + 1]], add=True)
                return _

            lax.fori_loop(0, HBLK // 2, blk, None)
        plsc.subcore_barrier()
        pltpu.sync_copy(acc_sh.at[pl.ds(sid * ROWS_PER_SUB, ROWS_PER_SUB)],
                        s_hbm.at[chunk, pl.ds(sid * ROWS_PER_SUB, ROWS_PER_SUB)])
        plsc.subcore_barrier()


def _sc_aggregate(g_chunks, rowidx4, colidx, zeros_b):
    g2 = g_chunks.reshape(NCH * NP, 128)
    return pl.kernel(
        _agg_body,
        out_type=jax.ShapeDtypeStruct((NCH, NP, 128), _f32),
        mesh=_sc_mesh(),
        scratch_types=[
            pltpu.VMEM((HBLK, EB), _i32),
            pltpu.VMEM((HBLK, EB), _i32),
            pltpu.VMEM((EB, 128), _f32),
            pltpu.VMEM((EB, 128), _f32),
            pltpu.SemaphoreType.DMA,
            pltpu.SemaphoreType.DMA,
            pltpu.VMEM_SHARED((NP, 128), _f32),
        ],
    )(g2, rowidx4, colidx, zeros_b)


def _dis_kernel(deg_ref, dis_ref):
    dis_ref[...] = lax.rsqrt(deg_ref[0] + deg_ref[1] + 1.0)


def _tc_dis(deg):
    return pl.pallas_call(
        _dis_kernel,
        grid=(NT,),
        in_specs=[pl.BlockSpec((NC, 512, 128), lambda n: (0, n, 0))],
        out_specs=pl.BlockSpec((512, 128), lambda n: (n, 0)),
        out_shape=jax.ShapeDtypeStruct((NP, 128), _f32),
    )(deg)


def _in_mm_kernel(x_ref, w_ref, b_ref, h_ref, acc_ref):
    kc = pl.program_id(1)

    @pl.when(kc == 0)
    def _():
        acc_ref[...] = jnp.zeros_like(acc_ref)

    acc_ref[...] += jnp.dot(x_ref[...], w_ref[...],
                            preferred_element_type=_f32)

    @pl.when(kc == (D_IN // 128) - 1)
    def _():
        h_ref[...] = acc_ref[...] + b_ref[0:1, :]


def _tc_in_matmul(xp, W_in, b_in2):
    return pl.pallas_call(
        _in_mm_kernel,
        grid=(NT, D_IN // 128),
        in_specs=[
            pl.BlockSpec((512, 128), lambda n, k: (n, k)),
            pl.BlockSpec((128, H), lambda n, k: (k, 0)),
            pl.BlockSpec((8, H), lambda n, k: (0, 0)),
        ],
        out_specs=pl.BlockSpec((512, H), lambda n, k: (n, 0)),
        out_shape=jax.ShapeDtypeStruct((NP, H), _f32),
        scratch_shapes=[pltpu.VMEM((512, H), _f32)],
        compiler_params=pltpu.CompilerParams(
            dimension_semantics=("parallel", "arbitrary")),
    )(xp, W_in, b_in2)


def _dual_mm_epilogue(dis_ref, g_ref, r_ref, accg_ref, accr_ref):
    dis_col = dis_ref[:, 0:1]
    for c in range(NCH):
        g_ref[c] = dis_col * accg_ref[:, c * 128:(c + 1) * 128]
    r_ref[...] = accr_ref[...]


def _dual_mm_kernel(h_ref, wc_ref, wr_ref, dis_ref, g_ref, r_ref,
                    accg_ref, accr_ref):
    kc = pl.program_id(1)

    @pl.when(kc == 0)
    def _():
        accg_ref[...] = jnp.zeros_like(accg_ref)
        accr_ref[...] = jnp.zeros_like(accr_ref)

    h_blk = h_ref[...]
    accg_ref[...] += jnp.dot(h_blk, wc_ref[...], preferred_element_type=_f32)
    accr_ref[...] += jnp.dot(h_blk, wr_ref[...], preferred_element_type=_f32)

    @pl.when(kc == NCH - 1)
    def _():
        _dual_mm_epilogue(dis_ref, g_ref, r_ref, accg_ref, accr_ref)


def _bn_relu_block(y_blk, stats_ref, bng_ref, bnb_ref):
    mean = stats_ref[0, 0:1, :] * (1.0 / N_NODES)
    var = stats_ref[0, 1:2, :] * (1.0 / N_NODES) - mean * mean
    inv = lax.rsqrt(var + EPS)
    return jnp.maximum((y_blk - mean) * (inv * bng_ref[0, 0:1, :])
                       + bnb_ref[0, 0:1, :], 0.0)


def _dual_mm_fused_kernel(y_ref, stats_ref, bng_ref, bnb_ref,
                          wc_ref, wr_ref, dis_ref, g_ref, r_ref,
                          accg_ref, accr_ref):
    kc = pl.program_id(1)

    @pl.when(kc == 0)
    def _():
        accg_ref[...] = jnp.zeros_like(accg_ref)
        accr_ref[...] = jnp.zeros_like(accr_ref)

    h_blk = _bn_relu_block(y_ref[...], stats_ref, bng_ref, bnb_ref)
    accg_ref[...] += jnp.dot(h_blk, wc_ref[...], preferred_element_type=_f32)
    accr_ref[...] += jnp.dot(h_blk, wr_ref[...], preferred_element_type=_f32)

    @pl.when(kc == NCH - 1)
    def _():
        _dual_mm_epilogue(dis_ref, g_ref, r_ref, accg_ref, accr_ref)


_DUAL_OUT = (jax.ShapeDtypeStruct((NCH, NP, 128), _f32),
             jax.ShapeDtypeStruct((NP, H), _f32))
_DUAL_OUT_SPECS = (pl.BlockSpec((NCH, 512, 128), lambda n, k: (0, n, 0)),
                   pl.BlockSpec((512, H), lambda n, k: (n, 0)))
_DUAL_SCRATCH = [pltpu.VMEM((512, H), _f32), pltpu.VMEM((512, H), _f32)]
_CP = pltpu.CompilerParams(dimension_semantics=("parallel", "arbitrary"))
_H_SPEC = pl.BlockSpec((512, 128), lambda n, k: (n, k))
_W_SPEC = pl.BlockSpec((128, H), lambda n, k: (k, 0))
_DIS_SPEC = pl.BlockSpec((512, 128), lambda n, k: (n, 0))
_ST_SPEC = pl.BlockSpec((1, 8, 128), lambda n, k: (k, 0, 0))


def _tc_dual_matmul(h, Wc, Wr, dis):
    return pl.pallas_call(
        _dual_mm_kernel, grid=(NT, NCH),
        in_specs=[_H_SPEC, _W_SPEC, _W_SPEC, _DIS_SPEC],
        out_specs=_DUAL_OUT_SPECS, out_shape=_DUAL_OUT,
        scratch_shapes=_DUAL_SCRATCH, compiler_params=_CP,
    )(h, Wc, Wr, dis)


def _tc_dual_matmul_fused(y, stats, bng, bnb, Wc, Wr, dis):
    return pl.pallas_call(
        _dual_mm_fused_kernel, grid=(NT, NCH),
        in_specs=[_H_SPEC, _ST_SPEC, _ST_SPEC, _ST_SPEC, _W_SPEC, _W_SPEC,
                  _DIS_SPEC],
        out_specs=_DUAL_OUT_SPECS, out_shape=_DUAL_OUT,
        scratch_shapes=_DUAL_SCRATCH, compiler_params=_CP,
    )(y, stats, bng, bnb, Wc, Wr, dis)


def _combine_kernel(s_ref, g_ref, r_ref, dis_ref, b_ref, y_ref, stats_ref):
    n = pl.program_id(0)

    @pl.when(n == 0)
    def _():
        stats_ref[...] = jnp.zeros_like(stats_ref)

    dis_col = dis_ref[:, 0:1]
    rowid = n * 512 + lax.broadcasted_iota(_i32, (512, 1), 0)
    valid = rowid < N_NODES
    for c in range(NCH):
        ycol = (s_ref[c] + g_ref[c]) * dis_col \
            + r_ref[:, c * 128:(c + 1) * 128] + b_ref[c, 0:1, :]
        ycol = jnp.where(valid, ycol, 0.0)
        y_ref[:, c * 128:(c + 1) * 128] = ycol
        stats_ref[c, 0, :] += jnp.sum(ycol, axis=0)
        stats_ref[c, 1, :] += jnp.sum(ycol * ycol, axis=0)


def _tc_combine(S, g, r, dis, conv_b3):
    return pl.pallas_call(
        _combine_kernel,
        grid=(NT,),
        in_specs=[
            pl.BlockSpec((NCH, 512, 128), lambda n: (0, n, 0)),
            pl.BlockSpec((NCH, 512, 128), lambda n: (0, n, 0)),
            pl.BlockSpec((512, H), lambda n: (n, 0)),
            pl.BlockSpec((512, 128), lambda n: (n, 0)),
            pl.BlockSpec((NCH, 8, 128), lambda n: (0, 0, 0)),
        ],
        out_specs=(pl.BlockSpec((512, H), lambda n: (n, 0)),
                   pl.BlockSpec((NCH, 8, 128), lambda n: (0, 0, 0))),
        out_shape=(jax.ShapeDtypeStruct((NP, H), _f32),
                   jax.ShapeDtypeStruct((NCH, 8, 128), _f32)),
    )(S, g, r, dis, conv_b3)


def _head_kernel(y_ref, stats_ref, bng_ref, bnb_ref, wo_ref, bo_ref,
                 out_ref, acc_ref):
    kc = pl.program_id(1)

    @pl.when(kc == 0)
    def _():
        acc_ref[...] = jnp.zeros_like(acc_ref)

    h_blk = _bn_relu_block(y_ref[...], stats_ref, bng_ref, bnb_ref)
    acc_ref[...] += jnp.dot(h_blk, wo_ref[...], preferred_element_type=_f32)

    @pl.when(kc == NCH - 1)
    def _():
        z = acc_ref[...] + bo_ref[0:1, :]
        m = jnp.max(z, axis=1, keepdims=True)
        lse = jnp.log(jnp.sum(jnp.exp(z - m), axis=1, keepdims=True))
        out_ref[...] = z - m - lse


def _tc_head(y, stats, bng, bnb, Wo, bo):
    return pl.pallas_call(
        _head_kernel,
        grid=(NT, NCH),
        in_specs=[
            pl.BlockSpec((512, 128), lambda n, k: (n, k)),
            pl.BlockSpec((1, 8, 128), lambda n, k: (k, 0, 0)),
            pl.BlockSpec((1, 8, 128), lambda n, k: (k, 0, 0)),
            pl.BlockSpec((1, 8, 128), lambda n, k: (k, 0, 0)),
            pl.BlockSpec((128, 128), lambda n, k: (k, 0)),
            pl.BlockSpec((8, 128), lambda n, k: (0, 0)),
        ],
        out_specs=pl.BlockSpec((512, 128), lambda n, k: (n, 0)),
        out_shape=jax.ShapeDtypeStruct((NP, 128), _f32),
        scratch_shapes=[pltpu.VMEM((512, 128), _f32)],
        compiler_params=pltpu.CompilerParams(
            dimension_semantics=("parallel", "arbitrary")),
    )(y, stats, bng, bnb, Wo, bo)


def kernel(x, edge_index, W_in, b_in, conv_W, conv_b, res_W, bn_g, bn_b,
           W_out, b_out):
    row = edge_index[0].astype(_i32)
    col = edge_index[1].astype(_i32)
    pad = E_PAD - N_EDGES
    rowp = jnp.concatenate([row, jnp.zeros((pad,), _i32)])
    colp = jnp.concatenate([col, jnp.full((pad,), N_NODES, _i32)])
    col_sh = colp.reshape(NS, NBLK, EB)
    col_sh2 = colp.reshape(NS, NBLK, EB)
    rowidx4 = (rowp.reshape(NS, NBLK, EB)[None]
               + (jnp.arange(NCH, dtype=_i32) * NP)[:, None, None, None])

    zeros_b = jnp.zeros((ROWS_PER_SUB, 128), _f32)
    ones_b = jnp.ones((EB, 128), _f32)

    deg = _sc_degree(col_sh, ones_b, zeros_b)
    dis = _tc_dis(deg)

    xp = jnp.pad(x, ((0, NP - N_NODES), (0, 0)))
    b_in2 = jnp.broadcast_to(b_in, (8, H))
    conv_b3 = jnp.broadcast_to(conv_b.reshape(N_LAYERS, NCH, 1, 128),
                               (N_LAYERS, NCH, 8, 128))
    bn_g3 = jnp.broadcast_to(bn_g.reshape(N_LAYERS, NCH, 1, 128),
                             (N_LAYERS, NCH, 8, 128))
    bn_b3 = jnp.broadcast_to(bn_b.reshape(N_LAYERS, NCH, 1, 128),
                             (N_LAYERS, NCH, 8, 128))
    Wo_pad = jnp.pad(W_out, ((0, 0), (0, 128 - C_OUT)))
    bo_pad = jnp.broadcast_to(
        jnp.concatenate([b_out, jnp.full((128 - C_OUT,), -1e30, _f32)]),
        (8, 128))

    h0 = _tc_in_matmul(xp, W_in, b_in2)

    y, stats = None, None
    for i in range(N_LAYERS):
        if i == 0:
            g, r = _tc_dual_matmul(h0, conv_W[0], res_W[0], dis)
        else:
            g, r = _tc_dual_matmul_fused(y, stats, bn_g3[i - 1], bn_b3[i - 1],
                                         conv_W[i], res_W[i], dis)
        S = _sc_aggregate(g, rowidx4, col_sh2, zeros_b)
        y, stats = _tc_combine(S, g, r, dis, conv_b3[i])

    out = _tc_head(y, stats, bn_g3[N_LAYERS - 1], bn_b3[N_LAYERS - 1],
                   Wo_pad, bo_pad)
    return out[:N_NODES, :C_OUT]


# speedup vs baseline: 1.0751x; 1.0004x over previous
import functools

import jax
import jax.numpy as jnp
from jax import lax
from jax.experimental import pallas as pl
from jax.experimental.pallas import tpu as pltpu
from jax.experimental.pallas import tpu_sc as plsc

N_NODES = 10000
N_EDGES = 160000
D_IN = 256
H = 512
C_OUT = 18
N_LAYERS = 9
EPS = 1e-5

NP = 10240
NT = NP // 512
NC = 2
NS = 16
EB = 128
NBLK = 80
E_PAD = NS * NBLK * EB
ROWS_PER_SUB = NP // NS
NCH = 4

_f32 = jnp.float32
_i32 = jnp.int32


def _sc_mesh():
    return plsc.VectorSubcoreMesh(core_axis_name="c", subcore_axis_name="s")


def _deg_body(colidx_hbm, ones_hbm, zeros_hbm, deg_hbm, col_v, buf_v, acc_sh):
    core = lax.axis_index("c")
    sid = lax.axis_index("s")
    pltpu.sync_copy(zeros_hbm, acc_sh.at[pl.ds(sid * ROWS_PER_SUB, ROWS_PER_SUB)])
    pltpu.sync_copy(colidx_hbm.at[sid], col_v)
    pltpu.sync_copy(ones_hbm, buf_v)
    plsc.subcore_barrier()

    half = NBLK // NC

    def blk(j, _):
        pltpu.sync_copy(buf_v, acc_sh.at[col_v.at[core * half + j]], add=True)
        return _

    lax.fori_loop(0, half, blk, None)
    plsc.subcore_barrier()
    pltpu.sync_copy(acc_sh.at[pl.ds(sid * ROWS_PER_SUB, ROWS_PER_SUB)],
                    deg_hbm.at[core, pl.ds(sid * ROWS_PER_SUB, ROWS_PER_SUB)])


def _sc_degree(colidx, ones_b, zeros_b):
    return pl.kernel(
        _deg_body,
        out_type=jax.ShapeDtypeStruct((NC, NP, 128), _f32),
        mesh=_sc_mesh(),
        scratch_types=[
            pltpu.VMEM((NBLK, EB), _i32),
            pltpu.VMEM((EB, 128), _f32),
            pltpu.VMEM_SHARED((NP, 128), _f32),
        ],
    )(colidx, ones_b, zeros_b)


HBLK = NBLK // 2


def _agg_body(g2_hbm, rowidx_hbm, colidx_hbm, zeros_hbm, s_hbm,
              row_v, col_v, buf0_v, buf1_v, gs0, gs1, acc_sh):
    core = lax.axis_index("c")
    sid = lax.axis_index("s")
    for p in range(2):
        chunk = core * 2 + p
        pltpu.sync_copy(zeros_hbm, acc_sh.at[pl.ds(sid * ROWS_PER_SUB, ROWS_PER_SUB)])
        plsc.subcore_barrier()

        for q in range(NBLK // HBLK):
            pltpu.sync_copy(rowidx_hbm.at[chunk, sid, pl.ds(q * HBLK, HBLK)],
                            row_v)
            pltpu.sync_copy(colidx_hbm.at[sid, pl.ds(q * HBLK, HBLK)], col_v)
            pltpu.async_copy(g2_hbm.at[row_v.at[0]], buf0_v, gs0)

            def blk(jj, _):
                j = 2 * jj
                pltpu.async_copy(g2_hbm.at[row_v.at[j + 1]], buf1_v, gs1)
                pltpu.make_async_copy(g2_hbm.at[row_v.at[j]], buf0_v,
                                      gs0).wait()
                pltpu.sync_copy(buf0_v, acc_sh.at[col_v.at[j]], add=True)

                @pl.when(jj + 1 < HBLK // 2)
                def _():
                    pltpu.async_copy(g2_hbm.at[row_v.at[j + 2]], buf0_v, gs0)

                pltpu.make_async_copy(g2_hbm.at[row_v.at[j + 1]], buf1_v,
                                      gs1).wait()
                pltpu.sync_copy(buf1_v, acc_sh.at[col_v.at[j + 1]], add=True)
                return _

            lax.fori_loop(0, HBLK // 2, blk, None)
        plsc.subcore_barrier()
        pltpu.sync_copy(acc_sh.at[pl.ds(sid * ROWS_PER_SUB, ROWS_PER_SUB)],
                        s_hbm.at[chunk, pl.ds(sid * ROWS_PER_SUB, ROWS_PER_SUB)])
        plsc.subcore_barrier()


def _sc_aggregate(g_chunks, rowidx4, colidx, zeros_b):
    g2 = g_chunks.reshape(NCH * NP, 128)
    return pl.kernel(
        _agg_body,
        out_type=jax.ShapeDtypeStruct((NCH, NP, 128), _f32),
        mesh=_sc_mesh(),
        scratch_types=[
            pltpu.VMEM((HBLK, EB), _i32),
            pltpu.VMEM((HBLK, EB), _i32),
            pltpu.VMEM((EB, 128), _f32),
            pltpu.VMEM((EB, 128), _f32),
            pltpu.SemaphoreType.DMA,
            pltpu.SemaphoreType.DMA,
            pltpu.VMEM_SHARED((NP, 128), _f32),
        ],
    )(g2, rowidx4, colidx, zeros_b)


def _dis_kernel(deg_ref, dis_ref):
    dis_ref[...] = lax.rsqrt(deg_ref[0] + deg_ref[1] + 1.0)


def _tc_dis(deg):
    return pl.pallas_call(
        _dis_kernel,
        grid=(NT,),
        in_specs=[pl.BlockSpec((NC, 512, 128), lambda n: (0, n, 0))],
        out_specs=pl.BlockSpec((512, 128), lambda n: (n, 0)),
        out_shape=jax.ShapeDtypeStruct((NP, 128), _f32),
    )(deg)


def _in_mm_kernel(x_ref, w_ref, b_ref, h_ref, acc_ref):
    kc = pl.program_id(1)

    @pl.when(kc == 0)
    def _():
        acc_ref[...] = jnp.zeros_like(acc_ref)

    acc_ref[...] += jnp.dot(x_ref[...], w_ref[...],
                            preferred_element_type=_f32)

    @pl.when(kc == (D_IN // 128) - 1)
    def _():
        h_ref[...] = acc_ref[...] + b_ref[0:1, :]


def _tc_in_matmul(xp, W_in, b_in2):
    return pl.pallas_call(
        _in_mm_kernel,
        grid=(NT, D_IN // 128),
        in_specs=[
            pl.BlockSpec((512, 128), lambda n, k: (n, k)),
            pl.BlockSpec((128, H), lambda n, k: (k, 0)),
            pl.BlockSpec((8, H), lambda n, k: (0, 0)),
        ],
        out_specs=pl.BlockSpec((512, H), lambda n, k: (n, 0)),
        out_shape=jax.ShapeDtypeStruct((NP, H), _f32),
        scratch_shapes=[pltpu.VMEM((512, H), _f32)],
        compiler_params=pltpu.CompilerParams(
            dimension_semantics=("parallel", "arbitrary")),
    )(xp, W_in, b_in2)


def _dual_mm_epilogue(dis_ref, g_ref, r_ref, accg_ref, accr_ref):
    dis_col = dis_ref[:, 0:1]
    for c in range(NCH):
        g_ref[c] = dis_col * accg_ref[:, c * 128:(c + 1) * 128]
    r_ref[...] = accr_ref[...]


def _dual_mm_kernel(h_ref, wc_ref, wr_ref, dis_ref, g_ref, r_ref,
                    accg_ref, accr_ref):
    kc = pl.program_id(1)

    @pl.when(kc == 0)
    def _():
        accg_ref[...] = jnp.zeros_like(accg_ref)
        accr_ref[...] = jnp.zeros_like(accr_ref)

    h_blk = h_ref[...].astype(jnp.bfloat16)
    accg_ref[...] += jnp.dot(h_blk, wc_ref[...].astype(jnp.bfloat16),
                             preferred_element_type=_f32)
    accr_ref[...] += jnp.dot(h_blk, wr_ref[...].astype(jnp.bfloat16),
                             preferred_element_type=_f32)

    @pl.when(kc == NCH - 1)
    def _():
        _dual_mm_epilogue(dis_ref, g_ref, r_ref, accg_ref, accr_ref)


def _bn_relu_block(y_blk, stats_ref, bng_ref, bnb_ref):
    mean = stats_ref[0, 0:1, :] * (1.0 / N_NODES)
    var = stats_ref[0, 1:2, :] * (1.0 / N_NODES) - mean * mean
    inv = lax.rsqrt(var + EPS)
    return jnp.maximum((y_blk - mean) * (inv * bng_ref[0, 0:1, :])
                       + bnb_ref[0, 0:1, :], 0.0)


def _dual_mm_fused_kernel(y_ref, stats_ref, bng_ref, bnb_ref,
                          wc_ref, wr_ref, dis_ref, g_ref, r_ref,
                          accg_ref, accr_ref):
    kc = pl.program_id(1)

    @pl.when(kc == 0)
    def _():
        accg_ref[...] = jnp.zeros_like(accg_ref)
        accr_ref[...] = jnp.zeros_like(accr_ref)

    h_blk = _bn_relu_block(y_ref[...], stats_ref, bng_ref,
                           bnb_ref).astype(jnp.bfloat16)
    accg_ref[...] += jnp.dot(h_blk, wc_ref[...].astype(jnp.bfloat16),
                             preferred_element_type=_f32)
    accr_ref[...] += jnp.dot(h_blk, wr_ref[...].astype(jnp.bfloat16),
                             preferred_element_type=_f32)

    @pl.when(kc == NCH - 1)
    def _():
        _dual_mm_epilogue(dis_ref, g_ref, r_ref, accg_ref, accr_ref)


_DUAL_OUT = (jax.ShapeDtypeStruct((NCH, NP, 128), _f32),
             jax.ShapeDtypeStruct((NP, H), _f32))
_DUAL_OUT_SPECS = (pl.BlockSpec((NCH, 512, 128), lambda n, k: (0, n, 0)),
                   pl.BlockSpec((512, H), lambda n, k: (n, 0)))
_DUAL_SCRATCH = [pltpu.VMEM((512, H), _f32), pltpu.VMEM((512, H), _f32)]
_CP = pltpu.CompilerParams(dimension_semantics=("parallel", "arbitrary"))
_H_SPEC = pl.BlockSpec((512, 128), lambda n, k: (n, k))
_W_SPEC = pl.BlockSpec((128, H), lambda n, k: (k, 0))
_DIS_SPEC = pl.BlockSpec((512, 128), lambda n, k: (n, 0))
_ST_SPEC = pl.BlockSpec((1, 8, 128), lambda n, k: (k, 0, 0))


def _tc_dual_matmul(h, Wc, Wr, dis):
    return pl.pallas_call(
        _dual_mm_kernel, grid=(NT, NCH),
        in_specs=[_H_SPEC, _W_SPEC, _W_SPEC, _DIS_SPEC],
        out_specs=_DUAL_OUT_SPECS, out_shape=_DUAL_OUT,
        scratch_shapes=_DUAL_SCRATCH, compiler_params=_CP,
    )(h, Wc, Wr, dis)


def _tc_dual_matmul_fused(y, stats, bng, bnb, Wc, Wr, dis):
    return pl.pallas_call(
        _dual_mm_fused_kernel, grid=(NT, NCH),
        in_specs=[_H_SPEC, _ST_SPEC, _ST_SPEC, _ST_SPEC, _W_SPEC, _W_SPEC,
                  _DIS_SPEC],
        out_specs=_DUAL_OUT_SPECS, out_shape=_DUAL_OUT,
        scratch_shapes=_DUAL_SCRATCH, compiler_params=_CP,
    )(y, stats, bng, bnb, Wc, Wr, dis)


def _combine_kernel(s_ref, g_ref, r_ref, dis_ref, b_ref, y_ref, stats_ref):
    n = pl.program_id(0)

    @pl.when(n == 0)
    def _():
        stats_ref[...] = jnp.zeros_like(stats_ref)

    dis_col = dis_ref[:, 0:1]
    rowid = n * 512 + lax.broadcasted_iota(_i32, (512, 1), 0)
    valid = rowid < N_NODES
    for c in range(NCH):
        ycol = (s_ref[c] + g_ref[c]) * dis_col \
            + r_ref[:, c * 128:(c + 1) * 128] + b_ref[c, 0:1, :]
        ycol = jnp.where(valid, ycol, 0.0)
        y_ref[:, c * 128:(c + 1) * 128] = ycol
        stats_ref[c, 0, :] += jnp.sum(ycol, axis=0)
        stats_ref[c, 1, :] += jnp.sum(ycol * ycol, axis=0)


def _tc_combine(S, g, r, dis, conv_b3):
    return pl.pallas_call(
        _combine_kernel,
        grid=(NT,),
        in_specs=[
            pl.BlockSpec((NCH, 512, 128), lambda n: (0, n, 0)),
            pl.BlockSpec((NCH, 512, 128), lambda n: (0, n, 0)),
            pl.BlockSpec((512, H), lambda n: (n, 0)),
            pl.BlockSpec((512, 128), lambda n: (n, 0)),
            pl.BlockSpec((NCH, 8, 128), lambda n: (0, 0, 0)),
        ],
        out_specs=(pl.BlockSpec((512, H), lambda n: (n, 0)),
                   pl.BlockSpec((NCH, 8, 128), lambda n: (0, 0, 0))),
        out_shape=(jax.ShapeDtypeStruct((NP, H), _f32),
                   jax.ShapeDtypeStruct((NCH, 8, 128), _f32)),
    )(S, g, r, dis, conv_b3)


def _head_kernel(y_ref, stats_ref, bng_ref, bnb_ref, wo_ref, bo_ref,
                 out_ref, acc_ref):
    kc = pl.program_id(1)

    @pl.when(kc == 0)
    def _():
        acc_ref[...] = jnp.zeros_like(acc_ref)

    h_blk = _bn_relu_block(y_ref[...], stats_ref, bng_ref, bnb_ref)
    acc_ref[...] += jnp.dot(h_blk, wo_ref[...], preferred_element_type=_f32)

    @pl.when(kc == NCH - 1)
    def _():
        z = acc_ref[...] + bo_ref[0:1, :]
        m = jnp.max(z, axis=1, keepdims=True)
        lse = jnp.log(jnp.sum(jnp.exp(z - m), axis=1, keepdims=True))
        out_ref[...] = z - m - lse


def _tc_head(y, stats, bng, bnb, Wo, bo):
    return pl.pallas_call(
        _head_kernel,
        grid=(NT, NCH),
        in_specs=[
            pl.BlockSpec((512, 128), lambda n, k: (n, k)),
            pl.BlockSpec((1, 8, 128), lambda n, k: (k, 0, 0)),
            pl.BlockSpec((1, 8, 128), lambda n, k: (k, 0, 0)),
            pl.BlockSpec((1, 8, 128), lambda n, k: (k, 0, 0)),
            pl.BlockSpec((128, 128), lambda n, k: (k, 0)),
            pl.BlockSpec((8, 128), lambda n, k: (0, 0)),
        ],
        out_specs=pl.BlockSpec((512, 128), lambda n, k: (n, 0)),
        out_shape=jax.ShapeDtypeStruct((NP, 128), _f32),
        scratch_shapes=[pltpu.VMEM((512, 128), _f32)],
        compiler_params=pltpu.CompilerParams(
            dimension_semantics=("parallel", "arbitrary")),
    )(y, stats, bng, bnb, Wo, bo)


def kernel(x, edge_index, W_in, b_in, conv_W, conv_b, res_W, bn_g, bn_b,
           W_out, b_out):
    row = edge_index[0].astype(_i32)
    col = edge_index[1].astype(_i32)
    pad = E_PAD - N_EDGES
    rowp = jnp.concatenate([row, jnp.zeros((pad,), _i32)])
    colp = jnp.concatenate([col, jnp.full((pad,), N_NODES, _i32)])
    col_sh = colp.reshape(NS, NBLK, EB)
    col_sh2 = colp.reshape(NS, NBLK, EB)
    rowidx4 = (rowp.reshape(NS, NBLK, EB)[None]
               + (jnp.arange(NCH, dtype=_i32) * NP)[:, None, None, None])

    zeros_b = jnp.zeros((ROWS_PER_SUB, 128), _f32)
    ones_b = jnp.ones((EB, 128), _f32)

    deg = _sc_degree(col_sh, ones_b, zeros_b)
    dis = _tc_dis(deg)

    xp = jnp.pad(x, ((0, NP - N_NODES), (0, 0)))
    b_in2 = jnp.broadcast_to(b_in, (8, H))
    conv_b3 = jnp.broadcast_to(conv_b.reshape(N_LAYERS, NCH, 1, 128),
                               (N_LAYERS, NCH, 8, 128))
    bn_g3 = jnp.broadcast_to(bn_g.reshape(N_LAYERS, NCH, 1, 128),
                             (N_LAYERS, NCH, 8, 128))
    bn_b3 = jnp.broadcast_to(bn_b.reshape(N_LAYERS, NCH, 1, 128),
                             (N_LAYERS, NCH, 8, 128))
    Wo_pad = jnp.pad(W_out, ((0, 0), (0, 128 - C_OUT)))
    bo_pad = jnp.broadcast_to(
        jnp.concatenate([b_out, jnp.full((128 - C_OUT,), -1e30, _f32)]),
        (8, 128))

    h0 = _tc_in_matmul(xp, W_in, b_in2)

    y, stats = None, None
    for i in range(N_LAYERS):
        if i == 0:
            g, r = _tc_dual_matmul(h0, conv_W[0], res_W[0], dis)
        else:
            g, r = _tc_dual_matmul_fused(y, stats, bn_g3[i - 1], bn_b3[i - 1],
                                         conv_W[i], res_W[i], dis)
        S = _sc_aggregate(g, rowidx4, col_sh2, zeros_b)
        y, stats = _tc_combine(S, g, r, dis, conv_b3[i])

    out = _tc_head(y, stats, bn_g3[N_LAYERS - 1], bn_b3[N_LAYERS - 1],
                   Wo_pad, bo_pad)
    return out[:N_NODES, :C_OUT]


# speedup vs baseline: 1.0754x; 1.0003x over previous
import functools

import jax
import jax.numpy as jnp
from jax import lax
from jax.experimental import pallas as pl
from jax.experimental.pallas import tpu as pltpu
from jax.experimental.pallas import tpu_sc as plsc

N_NODES = 10000
N_EDGES = 160000
D_IN = 256
H = 512
C_OUT = 18
N_LAYERS = 9
EPS = 1e-5

NP = 10240
NT = NP // 512
NC = 2
NS = 16
EB = 128
NBLK = 80
E_PAD = NS * NBLK * EB
ROWS_PER_SUB = NP // NS
NCH = 4

_f32 = jnp.float32
_i32 = jnp.int32


def _sc_mesh():
    return plsc.VectorSubcoreMesh(core_axis_name="c", subcore_axis_name="s")


def _deg_body(colidx_hbm, ones_hbm, zeros_hbm, deg_hbm, col_v, buf_v, acc_sh):
    core = lax.axis_index("c")
    sid = lax.axis_index("s")
    pltpu.sync_copy(zeros_hbm, acc_sh.at[pl.ds(sid * ROWS_PER_SUB, ROWS_PER_SUB)])
    pltpu.sync_copy(colidx_hbm.at[sid], col_v)
    pltpu.sync_copy(ones_hbm, buf_v)
    plsc.subcore_barrier()

    half = NBLK // NC

    def blk(j, _):
        pltpu.sync_copy(buf_v, acc_sh.at[col_v.at[core * half + j]], add=True)
        return _

    lax.fori_loop(0, half, blk, None)
    plsc.subcore_barrier()
    pltpu.sync_copy(acc_sh.at[pl.ds(sid * ROWS_PER_SUB, ROWS_PER_SUB)],
                    deg_hbm.at[core, pl.ds(sid * ROWS_PER_SUB, ROWS_PER_SUB)])


def _sc_degree(colidx, ones_b, zeros_b):
    return pl.kernel(
        _deg_body,
        out_type=jax.ShapeDtypeStruct((NC, NP, 128), _f32),
        mesh=_sc_mesh(),
        scratch_types=[
            pltpu.VMEM((NBLK, EB), _i32),
            pltpu.VMEM((EB, 128), _f32),
            pltpu.VMEM_SHARED((NP, 128), _f32),
        ],
    )(colidx, ones_b, zeros_b)


HBLK = NBLK // 2


def _agg_body(g2_hbm, rowidx_hbm, colidx_hbm, zeros_hbm, s_hbm,
              row_v, col_v, buf0_v, buf1_v, gs0, gs1, acc_sh):
    core = lax.axis_index("c")
    sid = lax.axis_index("s")
    for p in range(2):
        chunk = core * 2 + p
        pltpu.sync_copy(zeros_hbm, acc_sh.at[pl.ds(sid * ROWS_PER_SUB, ROWS_PER_SUB)])
        plsc.subcore_barrier()

        for q in range(NBLK // HBLK):
            pltpu.sync_copy(rowidx_hbm.at[chunk, sid, pl.ds(q * HBLK, HBLK)],
                            row_v)
            pltpu.sync_copy(colidx_hbm.at[sid, pl.ds(q * HBLK, HBLK)], col_v)
            pltpu.async_copy(g2_hbm.at[row_v.at[0]], buf0_v, gs0)

            def blk(jj, _):
                j = 2 * jj
                pltpu.async_copy(g2_hbm.at[row_v.at[j + 1]], buf1_v, gs1)
                pltpu.make_async_copy(g2_hbm.at[row_v.at[j]], buf0_v,
                                      gs0).wait()
                pltpu.sync_copy(buf0_v, acc_sh.at[col_v.at[j]], add=True)

                @pl.when(jj + 1 < HBLK // 2)
                def _():
                    pltpu.async_copy(g2_hbm.at[row_v.at[j + 2]], buf0_v, gs0)

                pltpu.make_async_copy(g2_hbm.at[row_v.at[j + 1]], buf1_v,
                                      gs1).wait()
                pltpu.sync_copy(buf1_v, acc_sh.at[col_v.at[j + 1]], add=True)
                return _

            lax.fori_loop(0, HBLK // 2, blk, None)
        plsc.subcore_barrier()
        pltpu.sync_copy(acc_sh.at[pl.ds(sid * ROWS_PER_SUB, ROWS_PER_SUB)],
                        s_hbm.at[chunk, pl.ds(sid * ROWS_PER_SUB, ROWS_PER_SUB)])
        plsc.subcore_barrier()


def _sc_aggregate(g_chunks, rowidx4, colidx, zeros_b):
    g2 = g_chunks.reshape(NCH * NP, 128)
    return pl.kernel(
        _agg_body,
        out_type=jax.ShapeDtypeStruct((NCH, NP, 128), _f32),
        mesh=_sc_mesh(),
        scratch_types=[
            pltpu.VMEM((HBLK, EB), _i32),
            pltpu.VMEM((HBLK, EB), _i32),
            pltpu.VMEM((EB, 128), _f32),
            pltpu.VMEM((EB, 128), _f32),
            pltpu.SemaphoreType.DMA,
            pltpu.SemaphoreType.DMA,
            pltpu.VMEM_SHARED((NP, 128), _f32),
        ],
    )(g2, rowidx4, colidx, zeros_b)


def _dis_kernel(deg_ref, dis_ref):
    dis_ref[...] = lax.rsqrt(deg_ref[0] + deg_ref[1] + 1.0)


def _tc_dis(deg):
    return pl.pallas_call(
        _dis_kernel,
        grid=(NT,),
        in_specs=[pl.BlockSpec((NC, 512, 128), lambda n: (0, n, 0))],
        out_specs=pl.BlockSpec((512, 128), lambda n: (n, 0)),
        out_shape=jax.ShapeDtypeStruct((NP, 128), _f32),
    )(deg)


def _in_mm_kernel(x_ref, w_ref, b_ref, h_ref, acc_ref):
    kc = pl.program_id(1)

    @pl.when(kc == 0)
    def _():
        acc_ref[...] = jnp.zeros_like(acc_ref)

    acc_ref[...] += jnp.dot(x_ref[...], w_ref[...],
                            preferred_element_type=_f32)

    @pl.when(kc == (D_IN // 128) - 1)
    def _():
        h_ref[...] = acc_ref[...] + b_ref[0:1, :]


def _tc_in_matmul(xp, W_in, b_in2):
    return pl.pallas_call(
        _in_mm_kernel,
        grid=(NT, D_IN // 128),
        in_specs=[
            pl.BlockSpec((512, 128), lambda n, k: (n, k)),
            pl.BlockSpec((128, H), lambda n, k: (k, 0)),
            pl.BlockSpec((8, H), lambda n, k: (0, 0)),
        ],
        out_specs=pl.BlockSpec((512, H), lambda n, k: (n, 0)),
        out_shape=jax.ShapeDtypeStruct((NP, H), _f32),
        scratch_shapes=[pltpu.VMEM((512, H), _f32)],
        compiler_params=pltpu.CompilerParams(
            dimension_semantics=("parallel", "arbitrary")),
    )(xp, W_in, b_in2)


def _dual_mm_epilogue(dis_ref, g_ref, r_ref, accg_ref, accr_ref):
    dis_col = dis_ref[:, 0:1]
    for c in range(NCH):
        g_ref[c] = dis_col * accg_ref[:, c * 128:(c + 1) * 128]
    r_ref[...] = accr_ref[...]


def _dual_mm_kernel(h_ref, wc_ref, wr_ref, dis_ref, g_ref, r_ref,
                    accg_ref, accr_ref):
    kc = pl.program_id(1)

    @pl.when(kc == 0)
    def _():
        accg_ref[...] = jnp.zeros_like(accg_ref)
        accr_ref[...] = jnp.zeros_like(accr_ref)

    h_blk = h_ref[...]
    accg_ref[...] += jnp.dot(h_blk, wc_ref[...], preferred_element_type=_f32)
    accr_ref[...] += jnp.dot(h_blk, wr_ref[...], preferred_element_type=_f32)

    @pl.when(kc == NCH - 1)
    def _():
        _dual_mm_epilogue(dis_ref, g_ref, r_ref, accg_ref, accr_ref)


def _bn_relu_block(y_blk, stats_ref, bng_ref, bnb_ref):
    mean = stats_ref[0, 0:1, :] * (1.0 / N_NODES)
    var = stats_ref[0, 1:2, :] * (1.0 / N_NODES) - mean * mean
    inv = lax.rsqrt(var + EPS)
    return jnp.maximum((y_blk - mean) * (inv * bng_ref[0, 0:1, :])
                       + bnb_ref[0, 0:1, :], 0.0)


def _dual_mm_fused_kernel(y_ref, stats_ref, bng_ref, bnb_ref,
                          wc_ref, wr_ref, dis_ref, g_ref, r_ref,
                          accg_ref, accr_ref):
    kc = pl.program_id(1)

    @pl.when(kc == 0)
    def _():
        accg_ref[...] = jnp.zeros_like(accg_ref)
        accr_ref[...] = jnp.zeros_like(accr_ref)

    h_blk = _bn_relu_block(y_ref[...], stats_ref, bng_ref, bnb_ref)
    accg_ref[...] += jnp.dot(h_blk, wc_ref[...], preferred_element_type=_f32)
    accr_ref[...] += jnp.dot(h_blk, wr_ref[...], preferred_element_type=_f32)

    @pl.when(kc == NCH - 1)
    def _():
        _dual_mm_epilogue(dis_ref, g_ref, r_ref, accg_ref, accr_ref)


_DUAL_OUT = (jax.ShapeDtypeStruct((NCH, NP, 128), _f32),
             jax.ShapeDtypeStruct((NP, H), _f32))
_DUAL_OUT_SPECS = (pl.BlockSpec((NCH, 512, 128), lambda n, k: (0, n, 0)),
                   pl.BlockSpec((512, H), lambda n, k: (n, 0)))
_DUAL_SCRATCH = [pltpu.VMEM((512, H), _f32), pltpu.VMEM((512, H), _f32)]
_CP = pltpu.CompilerParams(dimension_semantics=("parallel", "arbitrary"))
_H_SPEC = pl.BlockSpec((512, 128), lambda n, k: (n, k))
_W_SPEC = pl.BlockSpec((128, H), lambda n, k: (k, 0))
_DIS_SPEC = pl.BlockSpec((512, 128), lambda n, k: (n, 0))
_ST_SPEC = pl.BlockSpec((1, 8, 128), lambda n, k: (k, 0, 0))


def _tc_dual_matmul(h, Wc, Wr, dis):
    return pl.pallas_call(
        _dual_mm_kernel, grid=(NT, NCH),
        in_specs=[_H_SPEC, _W_SPEC, _W_SPEC, _DIS_SPEC],
        out_specs=_DUAL_OUT_SPECS, out_shape=_DUAL_OUT,
        scratch_shapes=_DUAL_SCRATCH, compiler_params=_CP,
    )(h, Wc, Wr, dis)


def _tc_dual_matmul_fused(y, stats, bng, bnb, Wc, Wr, dis):
    return pl.pallas_call(
        _dual_mm_fused_kernel, grid=(NT, NCH),
        in_specs=[_H_SPEC, _ST_SPEC, _ST_SPEC, _ST_SPEC, _W_SPEC, _W_SPEC,
                  _DIS_SPEC],
        out_specs=_DUAL_OUT_SPECS, out_shape=_DUAL_OUT,
        scratch_shapes=_DUAL_SCRATCH, compiler_params=_CP,
    )(y, stats, bng, bnb, Wc, Wr, dis)


def _combine_kernel(s_ref, g_ref, r_ref, dis_ref, b_ref, y_ref, stats_ref):
    n = pl.program_id(0)

    @pl.when(n == 0)
    def _():
        stats_ref[...] = jnp.zeros_like(stats_ref)

    dis_col = dis_ref[:, 0:1]
    rowid = n * 512 + lax.broadcasted_iota(_i32, (512, 1), 0)
    valid = rowid < N_NODES
    for c in range(NCH):
        ycol = (s_ref[c] + g_ref[c]) * dis_col \
            + r_ref[:, c * 128:(c + 1) * 128] + b_ref[c, 0:1, :]
        ycol = jnp.where(valid, ycol, 0.0)
        y_ref[:, c * 128:(c + 1) * 128] = ycol
        stats_ref[c, 0, :] += jnp.sum(ycol, axis=0)
        stats_ref[c, 1, :] += jnp.sum(ycol * ycol, axis=0)


def _tc_combine(S, g, r, dis, conv_b3):
    return pl.pallas_call(
        _combine_kernel,
        grid=(NT,),
        in_specs=[
            pl.BlockSpec((NCH, 512, 128), lambda n: (0, n, 0)),
            pl.BlockSpec((NCH, 512, 128), lambda n: (0, n, 0)),
            pl.BlockSpec((512, H), lambda n: (n, 0)),
            pl.BlockSpec((512, 128), lambda n: (n, 0)),
            pl.BlockSpec((NCH, 8, 128), lambda n: (0, 0, 0)),
        ],
        out_specs=(pl.BlockSpec((512, H), lambda n: (n, 0)),
                   pl.BlockSpec((NCH, 8, 128), lambda n: (0, 0, 0))),
        out_shape=(jax.ShapeDtypeStruct((NP, H), _f32),
                   jax.ShapeDtypeStruct((NCH, 8, 128), _f32)),
    )(S, g, r, dis, conv_b3)


def _head_kernel(y_ref, stats_ref, bng_ref, bnb_ref, wo_ref, bo_ref,
                 out_ref, acc_ref):
    kc = pl.program_id(1)

    @pl.when(kc == 0)
    def _():
        acc_ref[...] = jnp.zeros_like(acc_ref)

    h_blk = _bn_relu_block(y_ref[...], stats_ref, bng_ref, bnb_ref)
    acc_ref[...] += jnp.dot(h_blk, wo_ref[...], preferred_element_type=_f32)

    @pl.when(kc == NCH - 1)
    def _():
        z = acc_ref[...] + bo_ref[0:1, :]
        m = jnp.max(z, axis=1, keepdims=True)
        lse = jnp.log(jnp.sum(jnp.exp(z - m), axis=1, keepdims=True))
        out_ref[...] = z - m - lse


def _tc_head(y, stats, bng, bnb, Wo, bo):
    return pl.pallas_call(
        _head_kernel,
        grid=(NT, NCH),
        in_specs=[
            pl.BlockSpec((512, 128), lambda n, k: (n, k)),
            pl.BlockSpec((1, 8, 128), lambda n, k: (k, 0, 0)),
            pl.BlockSpec((1, 8, 128), lambda n, k: (k, 0, 0)),
            pl.BlockSpec((1, 8, 128), lambda n, k: (k, 0, 0)),
            pl.BlockSpec((128, 128), lambda n, k: (k, 0)),
            pl.BlockSpec((8, 128), lambda n, k: (0, 0)),
        ],
        out_specs=pl.BlockSpec((512, 128), lambda n, k: (n, 0)),
        out_shape=jax.ShapeDtypeStruct((NP, 128), _f32),
        scratch_shapes=[pltpu.VMEM((512, 128), _f32)],
        compiler_params=pltpu.CompilerParams(
            dimension_semantics=("parallel", "arbitrary")),
    )(y, stats, bng, bnb, Wo, bo)


def kernel(x, edge_index, W_in, b_in, conv_W, conv_b, res_W, bn_g, bn_b,
           W_out, b_out):
    row = edge_index[0].astype(_i32)
    col = edge_index[1].astype(_i32)
    pad = E_PAD - N_EDGES
    rowp = jnp.concatenate([row, jnp.zeros((pad,), _i32)])
    colp = jnp.concatenate([col, jnp.full((pad,), N_NODES, _i32)])
    col_sh = colp.reshape(NS, NBLK, EB)
    col_sh2 = colp.reshape(NS, NBLK, EB)
    rowidx4 = (rowp.reshape(NS, NBLK, EB)[None]
               + (jnp.arange(NCH, dtype=_i32) * NP)[:, None, None, None])

    zeros_b = jnp.zeros((ROWS_PER_SUB, 128), _f32)
    ones_b = jnp.ones((EB, 128), _f32)

    deg = _sc_degree(col_sh, ones_b, zeros_b)
    dis = _tc_dis(deg)

    xp = jnp.pad(x, ((0, NP - N_NODES), (0, 0)))
    b_in2 = jnp.broadcast_to(b_in, (8, H))
    conv_b3 = jnp.broadcast_to(conv_b.reshape(N_LAYERS, NCH, 1, 128),
                               (N_LAYERS, NCH, 8, 128))
    bn_g3 = jnp.broadcast_to(bn_g.reshape(N_LAYERS, NCH, 1, 128),
                             (N_LAYERS, NCH, 8, 128))
    bn_b3 = jnp.broadcast_to(bn_b.reshape(N_LAYERS, NCH, 1, 128),
                             (N_LAYERS, NCH, 8, 128))
    Wo_pad = jnp.pad(W_out, ((0, 0), (0, 128 - C_OUT)))
    bo_pad = jnp.broadcast_to(
        jnp.concatenate([b_out, jnp.full((128 - C_OUT,), -1e30, _f32)]),
        (8, 128))

    h0 = _tc_in_matmul(xp, W_in, b_in2)

    y, stats = None, None
    for i in range(N_LAYERS):
        if i == 0:
            g, r = _tc_dual_matmul(h0, conv_W[0], res_W[0], dis)
        else:
            g, r = _tc_dual_matmul_fused(y, stats, bn_g3[i - 1], bn_b3[i - 1],
                                         conv_W[i], res_W[i], dis)
        S = _sc_aggregate(g, rowidx4, col_sh2, zeros_b)
        y, stats = _tc_combine(S, g, r, dis, conv_b3[i])

    out = _tc_head(y, stats, bn_g3[N_LAYERS - 1], bn_b3[N_LAYERS - 1],
                   Wo_pad, bo_pad)
    return out[:N_NODES, :C_OUT]
